# baseline (speedup 1.0000x reference)
import functools

import jax
import jax.numpy as jnp
from jax import lax
from jax.experimental import pallas as pl
from jax.experimental.pallas import tpu as pltpu
from jax.experimental.pallas import tpu_sc as plsc

TOTAL_TOKENS = 204800
EMBED_DIM = 128

_NC = 2
_NS = 16
_NW = _NC * _NS
_PER_W = TOTAL_TOKENS // _NW
_CHUNK = 128
_NCHUNK = _PER_W // _CHUNK

_mesh = plsc.VectorSubcoreMesh(core_axis_name="c", subcore_axis_name="s")


_NBUF = 5
_HALF = _CHUNK // 4
_NPAIR = _NCHUNK // _NBUF


@functools.partial(
    pl.kernel,
    out_type=jax.ShapeDtypeStruct((TOTAL_TOKENS, EMBED_DIM), jnp.float32),
    mesh=_mesh,
    scratch_types=[
        pltpu.VMEM((_PER_W,), jnp.int32),
        pltpu.VMEM((_NBUF, _CHUNK, EMBED_DIM), jnp.float32),
        pltpu.SemaphoreType.DMA((_NBUF,)),
        pltpu.SemaphoreType.DMA((_NBUF,)),
        pltpu.SemaphoreType.DMA((_NBUF,)),
        pltpu.SemaphoreType.DMA((_NBUF,)),
        pltpu.SemaphoreType.DMA((_NBUF,)),
    ],
)
def _emb_lookup(data_hbm, w_hbm, out_hbm, idx_v, rows_v, gsem0, gsem1, gsem2, gsem3, osem):
    wid = lax.axis_index("s") * _NC + lax.axis_index("c")
    base = wid * _PER_W

    pltpu.sync_copy(data_hbm.at[pl.ds(base, _PER_W)], idx_v)

    def pair(p, _):
        for b in range(_NBUF):
            j = p * _NBUF + b

            @pl.when(p > 0)
            def _drain(b=b):
                pltpu.make_async_copy(
                    rows_v.at[b], out_hbm.at[pl.ds(0, _CHUNK)], osem.at[b]
                ).wait()

            for h in range(4):
                pltpu.make_async_copy(
                    w_hbm.at[idx_v.at[pl.ds(j * _CHUNK + h * _HALF, _HALF)]],
                    rows_v.at[b, pl.ds(h * _HALF, _HALF)],
                    (gsem0, gsem1, gsem2, gsem3)[h].at[b],
                ).start()

        for b in range(_NBUF):
            j = p * _NBUF + b
            for h in range(4):
                pltpu.make_async_copy(
                    w_hbm.at[idx_v.at[pl.ds(j * _CHUNK + h * _HALF, _HALF)]],
                    rows_v.at[b, pl.ds(h * _HALF, _HALF)],
                    (gsem0, gsem1, gsem2, gsem3)[h].at[b],
                ).wait()
                pltpu.make_async_copy(
                    rows_v.at[b, pl.ds(h * _HALF, _HALF)],
                    out_hbm.at[pl.ds(base + j * _CHUNK + h * _HALF, _HALF)],
                    osem.at[b],
                ).start()
        return 0

    lax.fori_loop(0, _NPAIR, pair, 0)

    for b in range(_NBUF):
        pltpu.make_async_copy(
            rows_v.at[b], out_hbm.at[pl.ds(0, _CHUNK)], osem.at[b]
        ).wait()


def kernel(data, batch_sizes, W):
    del batch_sizes
    return _emb_lookup(data, W)


# speedup vs baseline: 1.0220x; 1.0220x over previous
import functools

import jax
import jax.numpy as jnp
from jax import lax
from jax.experimental import pallas as pl
from jax.experimental.pallas import tpu as pltpu
from jax.experimental.pallas import tpu_sc as plsc

TOTAL_TOKENS = 204800
EMBED_DIM = 128

_NC = 2
_NS = 16
_NW = _NC * _NS
_PER_W = TOTAL_TOKENS // _NW
_CHUNK = 128
_NCHUNK = _PER_W // _CHUNK

_mesh = plsc.VectorSubcoreMesh(core_axis_name="c", subcore_axis_name="s")


_NBUF = 5
_HALF = _CHUNK // 2
_NPAIR = _NCHUNK // _NBUF


@functools.partial(
    pl.kernel,
    out_type=jax.ShapeDtypeStruct((TOTAL_TOKENS, EMBED_DIM), jnp.float32),
    mesh=_mesh,
    scratch_types=[
        pltpu.VMEM((_PER_W,), jnp.int32),
        pltpu.VMEM((_NBUF, _CHUNK, EMBED_DIM), jnp.float32),
        pltpu.SemaphoreType.DMA((_NBUF,)),
        pltpu.SemaphoreType.DMA((_NBUF,)),
        pltpu.SemaphoreType.DMA((_NBUF,)),
    ],
)
def _emb_lookup(data_hbm, w_hbm, out_hbm, idx_v, rows_v, gsem0, gsem1, osem):
    wid = lax.axis_index("s") * _NC + lax.axis_index("c")
    base = wid * _PER_W

    pltpu.sync_copy(data_hbm.at[pl.ds(base, _PER_W)], idx_v)

    def pair(p, _):
        for b in range(_NBUF):
            j = p * _NBUF + b

            @pl.when(p > 0)
            def _drain(b=b):
                pltpu.make_async_copy(
                    rows_v.at[b], out_hbm.at[pl.ds(0, _CHUNK)], osem.at[b]
                ).wait()

            for h in range(2):
                pltpu.make_async_copy(
                    w_hbm.at[idx_v.at[pl.ds(j * _CHUNK + h * _HALF, _HALF)]],
                    rows_v.at[b, pl.ds(h * _HALF, _HALF)],
                    (gsem0 if h == 0 else gsem1).at[b],
                ).start()

        for b in range(_NBUF):
            j = p * _NBUF + b
            for h in range(2):
                pltpu.make_async_copy(
                    w_hbm.at[idx_v.at[pl.ds(j * _CHUNK + h * _HALF, _HALF)]],
                    rows_v.at[b, pl.ds(h * _HALF, _HALF)],
                    (gsem0 if h == 0 else gsem1).at[b],
                ).wait()
                pltpu.make_async_copy(
                    rows_v.at[b, pl.ds(h * _HALF, _HALF)],
                    out_hbm.at[pl.ds(base + j * _CHUNK + h * _HALF, _HALF)],
                    osem.at[b],
                ).start()
        return 0

    lax.fori_loop(0, _NPAIR, pair, 0)

    for b in range(_NBUF):
        pltpu.make_async_copy(
            rows_v.at[b], out_hbm.at[pl.ds(0, _CHUNK)], osem.at[b]
        ).wait()


def kernel(data, batch_sizes, W):
    del batch_sizes
    return _emb_lookup(data, W)
